# baseline (speedup 1.0000x reference)
import jax
import jax.numpy as jnp
from jax.experimental import pallas as pl

_EMB = 16
_LANES = 128
_REP = _LANES // _EMB


def _fill_block(pat_ref, out_ref):
    out_ref[:, :] = jnp.broadcast_to(pat_ref[0:1, :], out_ref.shape)


def _pick_block(rows: int, cap: int = 25000) -> int:
    for b in range(min(cap, rows), 0, -1):
        if rows % b == 0:
            return b
    return rows


def kernel(edge_index, emb_table):
    E = edge_index.shape[1]
    total = E * _EMB
    if total % _LANES == 0:
        rows = total // _LANES
        pat = jnp.tile(emb_table[0], _REP)[None, :]
        block = _pick_block(rows)
        out = pl.pallas_call(
            _fill_block,
            grid=(rows // block,),
            in_specs=[pl.BlockSpec((1, _LANES), lambda i: (0, 0))],
            out_specs=pl.BlockSpec((block, _LANES), lambda i: (i, 0)),
            out_shape=jax.ShapeDtypeStruct((rows, _LANES), jnp.float32),
        )(pat)
        return out.reshape(E, _EMB)
    block = _pick_block(E)
    out = pl.pallas_call(
        _fill_block,
        grid=(E // block,),
        in_specs=[pl.BlockSpec((1, _EMB), lambda i: (0, 0))],
        out_specs=pl.BlockSpec((block, _EMB), lambda i: (i, 0)),
        out_shape=jax.ShapeDtypeStruct((E, _EMB), jnp.float32),
    )(emb_table)
    return out
